# baseline (speedup 1.0000x reference)
import functools

import jax
import jax.numpy as jnp
from jax.experimental import pallas as pl
from jax.experimental.pallas import tpu as pltpu

HIDDEN = 1024
INTER = 2048
E = 8
TOPK = 2

TM = 256
TMR = 512


def _router_body(x_ref, rw_ref, logits_ref):
    x = x_ref[...]
    rw = rw_ref[...]
    logits_ref[...] = jax.lax.dot_general(
        x, rw, (((1,), (1,)), ((), ())),
        preferred_element_type=jnp.float32)


def _router_logits(x, router_w):
    T = x.shape[0]
    return pl.pallas_call(
        _router_body,
        grid=(T // TMR,),
        in_specs=[
            pl.BlockSpec((TMR, HIDDEN), lambda i: (i, 0)),
            pl.BlockSpec((E, HIDDEN), lambda i: (0, 0)),
        ],
        out_specs=pl.BlockSpec((TMR, E), lambda i: (i, 0)),
        out_shape=jax.ShapeDtypeStruct((T, E), jnp.float32),
    )(x, router_w)


def _gemm_body(tile_expert_ref, xg_ref, w1_ref, w2_ref, w3_ref, gates_ref,
               yg_ref):
    i = pl.program_id(0)

    @pl.when(tile_expert_ref[i] < E)
    def _():
        x = xg_ref[...]
        w1 = w1_ref[0]
        w2 = w2_ref[0]
        w3 = w3_ref[0]
        h1 = jnp.dot(x, w1, preferred_element_type=jnp.float32)
        h2 = jnp.dot(x, w2, preferred_element_type=jnp.float32)
        h = (jax.nn.silu(h1) * h2).astype(jnp.bfloat16)
        y = jnp.dot(h, w3, preferred_element_type=jnp.float32)
        g = gates_ref[0, 0, :]
        yg_ref[...] = y * g[:, None]


def _grouped_gemm(xg, w1, w2, w3, gates_tiles, tile_expert, nt):
    PP = xg.shape[0]
    grid_spec = pltpu.PrefetchScalarGridSpec(
        num_scalar_prefetch=1,
        grid=(nt,),
        in_specs=[
            pl.BlockSpec((TM, HIDDEN), lambda i, te: (i, 0)),
            pl.BlockSpec((1, HIDDEN, INTER),
                         lambda i, te: (jnp.minimum(te[i], E - 1), 0, 0)),
            pl.BlockSpec((1, HIDDEN, INTER),
                         lambda i, te: (jnp.minimum(te[i], E - 1), 0, 0)),
            pl.BlockSpec((1, INTER, HIDDEN),
                         lambda i, te: (jnp.minimum(te[i], E - 1), 0, 0)),
            pl.BlockSpec((1, 1, TM), lambda i, te: (i, 0, 0)),
        ],
        out_specs=pl.BlockSpec((TM, HIDDEN), lambda i, te: (i, 0)),
    )
    return pl.pallas_call(
        _gemm_body,
        grid_spec=grid_spec,
        out_shape=jax.ShapeDtypeStruct((PP, HIDDEN), jnp.float32),
    )(tile_expert, xg, w1, w2, w3, gates_tiles)


def kernel(hidden_states, router_w, w1, w2, w3):
    orig_shape = hidden_states.shape
    x = hidden_states.reshape(-1, HIDDEN)
    T = x.shape[0]
    P = T * TOPK
    NT = P // TM + E
    PP = NT * TM

    router_logits = _router_logits(x, router_w)

    probs = jax.nn.softmax(router_logits, axis=-1)
    topw, sel = jax.lax.top_k(probs, TOPK)
    topw = topw / topw.sum(axis=-1, keepdims=True)

    e_flat = sel.reshape(-1)
    onehot = (e_flat[:, None] == jnp.arange(E)[None, :]).astype(jnp.int32)
    counts = onehot.sum(axis=0)
    rank = jnp.take_along_axis(jnp.cumsum(onehot, axis=0) - 1,
                               e_flat[:, None], axis=1)[:, 0]
    tiles_e = (counts + TM - 1) // TM
    cum_tiles = jnp.cumsum(tiles_e)
    astart = TM * (cum_tiles - tiles_e)
    pos = astart[e_flat] + rank
    slot_token = jnp.zeros((PP,), jnp.int32).at[pos].set(
        jnp.arange(P, dtype=jnp.int32) // TOPK)
    gates_slot = jnp.zeros((PP,), jnp.float32).at[pos].set(topw.reshape(-1))
    tile_expert = jnp.searchsorted(
        cum_tiles, jnp.arange(NT, dtype=jnp.int32),
        side="right").astype(jnp.int32)

    xg = jnp.take(x.astype(jnp.bfloat16), slot_token, axis=0)

    yg = _grouped_gemm(xg, w1.astype(jnp.bfloat16), w2.astype(jnp.bfloat16),
                       w3.astype(jnp.bfloat16),
                       gates_slot.reshape(NT, 1, TM), tile_expert, NT)

    pos2 = pos.reshape(T, TOPK)
    out = jnp.take(yg, pos2[:, 0], axis=0) + jnp.take(yg, pos2[:, 1], axis=0)
    return out.reshape(orig_shape), router_logits


# speedup vs baseline: 1.6656x; 1.6656x over previous
import functools

import jax
import jax.numpy as jnp
from jax import lax
from jax.experimental import pallas as pl
from jax.experimental.pallas import tpu as pltpu
from jax.experimental.pallas import tpu_sc as plsc

HIDDEN = 1024
INTER = 2048
E = 8
TOPK = 2

TM = 256
TMR = 512

NC = 2
NS = 16
NW = NC * NS
CHUNK = 64


def _router_body(x_ref, rw_ref, logits_ref):
    logits_ref[...] = jax.lax.dot_general(
        x_ref[...], rw_ref[...], (((1,), (1,)), ((), ())),
        preferred_element_type=jnp.float32)


def _router_logits(x, router_w):
    T = x.shape[0]
    return pl.pallas_call(
        _router_body,
        grid=(T // TMR,),
        in_specs=[
            pl.BlockSpec((TMR, HIDDEN), lambda i: (i, 0)),
            pl.BlockSpec((E, HIDDEN), lambda i: (0, 0)),
        ],
        out_specs=pl.BlockSpec((TMR, E), lambda i: (i, 0)),
        out_shape=jax.ShapeDtypeStruct((T, E), jnp.float32),
    )(x, router_w)


def _sc_scatter_rows(x, pos0, pos1, pp):
    T = x.shape[0]
    tpw = T // NW
    nchunk = tpw // CHUNK
    pos0r = pos0.reshape(NW, nchunk, CHUNK)
    pos1r = pos1.reshape(NW, nchunk, CHUNK)
    mesh = plsc.VectorSubcoreMesh(core_axis_name="c", subcore_axis_name="s")

    @functools.partial(
        pl.kernel, mesh=mesh,
        out_type=jax.ShapeDtypeStruct((pp, HIDDEN), jnp.float32),
        scratch_types=[
            pltpu.VMEM((nchunk, CHUNK), jnp.int32),
            pltpu.VMEM((nchunk, CHUNK), jnp.int32),
            pltpu.VMEM((CHUNK, HIDDEN), jnp.float32),
        ],
    )
    def k(x_hbm, p0_hbm, p1_hbm, xg_hbm, idx0_v, idx1_v, rows_v):
        wid = lax.axis_index("s") * NC + lax.axis_index("c")
        base = wid * tpw
        pltpu.sync_copy(p0_hbm.at[wid], idx0_v)
        pltpu.sync_copy(p1_hbm.at[wid], idx1_v)
        for c in range(nchunk):
            pltpu.sync_copy(x_hbm.at[pl.ds(base + c * CHUNK, CHUNK)], rows_v)
            pltpu.sync_copy(rows_v, xg_hbm.at[idx0_v.at[c]])
            pltpu.sync_copy(rows_v, xg_hbm.at[idx1_v.at[c]])

    return k(x, pos0r, pos1r)


def _sc_gather_pairs(yg, pos0, pos1):
    T = pos0.shape[0]
    tpw = T // NW
    nchunk = tpw // CHUNK
    pos0r = pos0.reshape(NW, nchunk, CHUNK)
    pos1r = pos1.reshape(NW, nchunk, CHUNK)
    mesh = plsc.VectorSubcoreMesh(core_axis_name="c", subcore_axis_name="s")
    row_ty = jax.ShapeDtypeStruct((T, HIDDEN), jnp.float32)

    @functools.partial(
        pl.kernel, mesh=mesh,
        out_type=(row_ty, row_ty),
        scratch_types=[
            pltpu.VMEM((nchunk, CHUNK), jnp.int32),
            pltpu.VMEM((nchunk, CHUNK), jnp.int32),
            pltpu.VMEM((CHUNK, HIDDEN), jnp.float32),
        ],
    )
    def k(yg_hbm, p0_hbm, p1_hbm, y0_hbm, y1_hbm, idx0_v, idx1_v, rows_v):
        wid = lax.axis_index("s") * NC + lax.axis_index("c")
        base = wid * tpw
        pltpu.sync_copy(p0_hbm.at[wid], idx0_v)
        pltpu.sync_copy(p1_hbm.at[wid], idx1_v)
        for c in range(nchunk):
            pltpu.sync_copy(yg_hbm.at[idx0_v.at[c]], rows_v)
            pltpu.sync_copy(rows_v, y0_hbm.at[pl.ds(base + c * CHUNK, CHUNK)])
            pltpu.sync_copy(yg_hbm.at[idx1_v.at[c]], rows_v)
            pltpu.sync_copy(rows_v, y1_hbm.at[pl.ds(base + c * CHUNK, CHUNK)])

    return k(yg, pos0r, pos1r)


def _gemm_body(tile_expert_ref, xg_ref, w1_ref, w2_ref, w3_ref, yg_ref):
    i = pl.program_id(0)

    @pl.when(tile_expert_ref[i] < E)
    def _():
        x = xg_ref[...]
        h1 = jnp.dot(x, w1_ref[0], preferred_element_type=jnp.float32)
        h2 = jnp.dot(x, w2_ref[0], preferred_element_type=jnp.float32)
        h = jax.nn.silu(h1) * h2
        yg_ref[...] = jnp.dot(h, w3_ref[0], preferred_element_type=jnp.float32)


def _grouped_gemm(xg, w1, w2, w3, tile_expert, nt):
    PP = xg.shape[0]
    grid_spec = pltpu.PrefetchScalarGridSpec(
        num_scalar_prefetch=1,
        grid=(nt,),
        in_specs=[
            pl.BlockSpec((TM, HIDDEN), lambda i, te: (i, 0)),
            pl.BlockSpec((1, HIDDEN, INTER),
                         lambda i, te: (jnp.minimum(te[i], E - 1), 0, 0)),
            pl.BlockSpec((1, HIDDEN, INTER),
                         lambda i, te: (jnp.minimum(te[i], E - 1), 0, 0)),
            pl.BlockSpec((1, INTER, HIDDEN),
                         lambda i, te: (jnp.minimum(te[i], E - 1), 0, 0)),
        ],
        out_specs=pl.BlockSpec((TM, HIDDEN), lambda i, te: (i, 0)),
    )
    return pl.pallas_call(
        _gemm_body,
        grid_spec=grid_spec,
        out_shape=jax.ShapeDtypeStruct((PP, HIDDEN), jnp.float32),
    )(tile_expert, xg, w1, w2, w3)


def kernel(hidden_states, router_w, w1, w2, w3):
    orig_shape = hidden_states.shape
    x = hidden_states.reshape(-1, HIDDEN)
    T = x.shape[0]
    P = T * TOPK
    NT = P // TM + E
    PP = NT * TM

    router_logits = _router_logits(x, router_w)

    probs = jax.nn.softmax(router_logits, axis=-1)
    topw, sel = jax.lax.top_k(probs, TOPK)
    topw = topw / topw.sum(axis=-1, keepdims=True)

    e_flat = sel.reshape(-1)
    onehot = (e_flat[:, None] == jnp.arange(E)[None, :]).astype(jnp.int32)
    counts = onehot.sum(axis=0)
    rank = jnp.take_along_axis(jnp.cumsum(onehot, axis=0) - 1,
                               e_flat[:, None], axis=1)[:, 0]
    tiles_e = (counts + TM - 1) // TM
    cum_tiles = jnp.cumsum(tiles_e)
    astart = TM * (cum_tiles - tiles_e)
    pos = (astart[e_flat] + rank).reshape(T, TOPK)
    pos0 = pos[:, 0]
    pos1 = pos[:, 1]
    tile_expert = jnp.searchsorted(
        cum_tiles, jnp.arange(NT, dtype=jnp.int32),
        side="right").astype(jnp.int32)

    xg = _sc_scatter_rows(x, pos0, pos1, PP)
    yg = _grouped_gemm(xg, w1, w2, w3, tile_expert, NT)

    y0, y1 = _sc_gather_pairs(yg, pos0, pos1)
    out = topw[:, 0, None] * y0 + topw[:, 1, None] * y1
    return out.reshape(orig_shape), router_logits
